# baseline (speedup 1.0000x reference)
import functools

import jax
import jax.numpy as jnp
from jax import lax
from jax.experimental import pallas as pl
from jax.experimental.pallas import tpu as pltpu
from jax.experimental.pallas import tpu_sc as plsc

B, D = 16384, 128
G, K = 12, 12
NC, NS, L = 2, 16, 16
NW = NC * NS
RPW = B // NW
BLK = L
NBLK = RPW // BLK

_GATHER_DNUMS = lax.GatherDimensionNumbers(
    offset_dims=(), collapsed_slice_dims=(0,), start_index_map=(0,))


def _vperm(vec, perm):
    return lax.gather(vec, perm.reshape(L, 1), _GATHER_DNUMS, (1,),
                      mode=lax.GatherScatterMode.PROMISE_IN_BOUNDS)


_mesh = plsc.VectorSubcoreMesh(
    core_axis_name="c", subcore_axis_name="s", num_cores=NC, num_subcores=NS)


@functools.partial(
    pl.kernel,
    out_type=jax.ShapeDtypeStruct((G, B), jnp.float32),
    mesh=_mesh,
    compiler_params=pltpu.CompilerParams(
        needs_layout_passes=False, use_tc_tiling_on_sc=False,
        disable_bounds_checks=True),
    scratch_types=[
        pltpu.VMEM((RPW * D,), jnp.float32),
        pltpu.VMEM((G * L,), jnp.int32),
        pltpu.VMEM((G, RPW), jnp.float32),
    ],
)
def _agg(inp_hbm, idx_hbm, out_hbm, rows_flat, idx_v, out_v):
    wid = lax.axis_index("s") * NC + lax.axis_index("c")
    base = wid * RPW

    pltpu.sync_copy(idx_hbm, idx_v)
    pltpu.sync_copy(inp_hbm.at[pl.ds(base * D, RPW * D)], rows_flat)

    iota = lax.broadcasted_iota(jnp.int32, (L,), 0)
    row_off = iota * D
    rots = [((iota + j) % K).astype(jnp.int32) for j in range(K)]

    for g in range(G):
        idx_row = idx_v[pl.ds(g * L, L)]
        addrs = [row_off + _vperm(idx_row, rots[j]) for j in range(K)]

        @plsc.parallel_loop(0, NBLK)
        def gmax_body(b, addrs=addrs, g=g):
            blk = rows_flat.at[pl.ds(b * (BLK * D), BLK * D)]
            m = plsc.load_gather(blk, [addrs[0]])
            for j in range(1, K):
                m = jnp.maximum(m, plsc.load_gather(blk, [addrs[j]]))
            out_v[g, pl.ds(b * BLK, BLK)] = m

    @plsc.parallel_loop(0, NBLK)
    def smax_body(b):
        maxes = [out_v[g, pl.ds(b * BLK, BLK)] for g in range(G)]
        mx = functools.reduce(jnp.maximum, maxes)
        exps = [jnp.exp(m - mx) for m in maxes]
        inv = 1.0 / functools.reduce(lambda a, c: a + c, exps)
        for g in range(G):
            out_v[g, pl.ds(b * BLK, BLK)] = exps[g] * inv

    pltpu.sync_copy(out_v, out_hbm.at[:, pl.ds(base, RPW)])


def kernel(inputs, subclass_indices):
    idx_pad = jnp.pad(subclass_indices, ((0, 0), (0, L - K)))
    return _agg(inputs.reshape(B * D), idx_pad.reshape(G * L)).T
